# Initial kernel scaffold; baseline (speedup 1.0000x reference)
import jax
import jax.numpy as jnp
from jax.experimental import pallas as pl


def kernel(seq, W):
    raise NotImplementedError("write your pallas kernel here")


# speedup vs baseline: 8.7032x; 8.7032x over previous
import jax
import jax.numpy as jnp
from jax import lax
from jax.experimental import pallas as pl
from jax.experimental.pallas import tpu as pltpu
from jax.experimental.pallas import tpu_sc as plsc

EMB = 64
B, LSEQ, T = 1024, 50, 20
NC, NS, LANES = 2, 16, 16
NW = NC * NS
ROWS = B * LSEQ
RPW = ROWS // NW
G = 32
ITERS = RPW // G
IDX_PER_ITER = G * T
IDX_CHUNK = 128
NSUB = IDX_PER_ITER // IDX_CHUNK


def _sc_body(seq_hbm, w_hbm, out_hbm, idx_v, rows_v, out_v, sem):
    wid = lax.axis_index("s") * NC + lax.axis_index("c")

    def outer(i, carry):
        blk = wid * ITERS + i
        pltpu.sync_copy(seq_hbm.at[blk], idx_v)
        copies = [
            pltpu.async_copy(
                w_hbm.at[idx_v.at[j]],
                rows_v.at[pl.ds(j * IDX_CHUNK, IDX_CHUNK)],
                sem,
            )
            for j in range(NSUB)
        ]
        for cp in copies:
            cp.wait()

        def group(g, c2):
            base = g * T
            for e in range(EMB // LANES):
                sl = pl.ds(e * LANES, LANES)
                s = jnp.zeros((LANES,), jnp.float32)
                c = jnp.zeros((LANES,), jnp.float32)
                for t in range(T):
                    r = rows_v[base + t, sl]
                    s = s + r
                    c = c + jnp.where(r != 0.0, 1.0, 0.0)
                out_v[g, sl] = jnp.where(c == 0.0, 0.0, s / c)
            return c2

        lax.fori_loop(0, G, group, 0)
        pltpu.sync_copy(out_v, out_hbm.at[pl.ds(blk * G, G)])
        return carry

    lax.fori_loop(0, ITERS, outer, 0)


def kernel(seq, W):
    w_full = jnp.concatenate([jnp.zeros((1, EMB), W.dtype), W], axis=0)
    seq3 = seq.reshape(ROWS // G, NSUB, IDX_CHUNK)
    mesh = plsc.VectorSubcoreMesh(core_axis_name="c", subcore_axis_name="s")
    out = pl.kernel(
        _sc_body,
        mesh=mesh,
        compiler_params=pltpu.CompilerParams(use_tc_tiling_on_sc=False),
        out_type=jax.ShapeDtypeStruct((ROWS, EMB), jnp.float32),
        scratch_types=[
            pltpu.VMEM((NSUB, IDX_CHUNK), jnp.int32),
            pltpu.VMEM((IDX_PER_ITER, EMB), jnp.float32),
            pltpu.VMEM((G, EMB), jnp.float32),
            pltpu.SemaphoreType.DMA,
        ],
    )(seq3, w_full)
    return out.reshape(B, LSEQ, EMB)


# speedup vs baseline: 11.7488x; 1.3499x over previous
import jax
import jax.numpy as jnp
from jax import lax
from jax.experimental import pallas as pl
from jax.experimental.pallas import tpu as pltpu
from jax.experimental.pallas import tpu_sc as plsc

EMB = 64
B, LSEQ, T = 1024, 50, 20
NC, NS, LANES = 2, 16, 16
NW = NC * NS
ROWS = B * LSEQ
RPW = ROWS // NW
G = 32
ITERS = RPW // G
IDX_PER_ITER = G * T
IDX_CHUNK = 128
NSUB = IDX_PER_ITER // IDX_CHUNK
NPAIR = ITERS // 2


def _sc_body(seq_hbm, w_hbm, out_hbm,
             idx0, idx1, rows0, rows1, out0, out1,
             sem0, sem1, osem0, osem1):
    wid = lax.axis_index("s") * NC + lax.axis_index("c")

    bufs = ((idx0, rows0, out0, sem0, osem0),
            (idx1, rows1, out1, sem1, osem1))

    def stage(i, buf):
        idx_v, rows_v, _, sem, _ = bufs[buf]
        pltpu.sync_copy(seq_hbm.at[wid * ITERS + i], idx_v)
        for j in range(NSUB):
            pltpu.async_copy(
                w_hbm.at[idx_v.at[j]],
                rows_v.at[pl.ds(j * IDX_CHUNK, IDX_CHUNK)],
                sem,
            )

    def drain(buf):
        idx_v, rows_v, _, sem, _ = bufs[buf]
        for j in range(NSUB):
            pltpu.make_async_copy(
                w_hbm.at[idx_v.at[j]],
                rows_v.at[pl.ds(j * IDX_CHUNK, IDX_CHUNK)],
                sem,
            ).wait()

    def compute(i, buf, first):
        idx_v, rows_v, out_v, _, osem = bufs[buf]
        blk = wid * ITERS + i
        @pl.when(jnp.logical_not(first))
        def _():
            pltpu.make_async_copy(
                out_v, out_hbm.at[pl.ds((blk - 2) * G, G)], osem
            ).wait()

        def group(g, c2):
            base = g * T
            for e in range(EMB // LANES):
                sl = pl.ds(e * LANES, LANES)
                s = jnp.zeros((LANES,), jnp.float32)
                c = jnp.zeros((LANES,), jnp.uint32)
                for t in range(T):
                    r = rows_v[base + t, sl]
                    s = s + r
                    c = c + jnp.minimum(
                        lax.bitcast_convert_type(r, jnp.uint32), 1)
                cf = c.astype(jnp.float32)
                out_v[g, sl] = jnp.where(c == 0, 0.0, s / cf)
            return c2

        lax.fori_loop(0, G, group, 0)
        pltpu.async_copy(out_v, out_hbm.at[pl.ds(blk * G, G)], osem)

    stage(0, 0)
    stage(1, 1)

    def outer(io, carry):
        i0 = 2 * io
        drain(0)
        compute(i0, 0, first=io == 0)

        @pl.when(io < NPAIR - 1)
        def _():
            stage(i0 + 2, 0)

        drain(1)
        compute(i0 + 1, 1, first=io == 0)

        @pl.when(io < NPAIR - 1)
        def _():
            stage(i0 + 3, 1)

        return carry

    lax.fori_loop(0, NPAIR, outer, 0)
    for buf in range(2):
        _, _, out_v, _, osem = bufs[buf]
        pltpu.make_async_copy(
            out_v,
            out_hbm.at[pl.ds((wid * ITERS + ITERS - 2 + buf) * G, G)],
            osem,
        ).wait()


def kernel(seq, W):
    w_full = jnp.concatenate([jnp.zeros((1, EMB), W.dtype), W], axis=0)
    seq3 = seq.reshape(ROWS // G, NSUB, IDX_CHUNK)
    mesh = plsc.VectorSubcoreMesh(core_axis_name="c", subcore_axis_name="s")
    out = pl.kernel(
        _sc_body,
        mesh=mesh,
        compiler_params=pltpu.CompilerParams(use_tc_tiling_on_sc=False),
        out_type=jax.ShapeDtypeStruct((ROWS, EMB), jnp.float32),
        scratch_types=[
            pltpu.VMEM((NSUB, IDX_CHUNK), jnp.int32),
            pltpu.VMEM((NSUB, IDX_CHUNK), jnp.int32),
            pltpu.VMEM((IDX_PER_ITER, EMB), jnp.float32),
            pltpu.VMEM((IDX_PER_ITER, EMB), jnp.float32),
            pltpu.VMEM((G, EMB), jnp.float32),
            pltpu.VMEM((G, EMB), jnp.float32),
            pltpu.SemaphoreType.DMA,
            pltpu.SemaphoreType.DMA,
            pltpu.SemaphoreType.DMA,
            pltpu.SemaphoreType.DMA,
        ],
    )(seq3, w_full)
    return out.reshape(B, LSEQ, EMB)


# speedup vs baseline: 13.9238x; 1.1851x over previous
import jax
import jax.numpy as jnp
from jax import lax
from jax.experimental import pallas as pl
from jax.experimental.pallas import tpu as pltpu
from jax.experimental.pallas import tpu_sc as plsc

EMB = 64
B, LSEQ, T = 1024, 50, 20
NC, NS, LANES = 2, 16, 16
NW = NC * NS
ROWS = B * LSEQ
RPW = ROWS // NW
G = 32
ITERS = RPW // G
IDX_PER_ITER = G * T
IDX_CHUNK = 128
NSUB = IDX_PER_ITER // IDX_CHUNK
NPAIR = ITERS // 2


def _sc_body(seq_hbm, w_hbm, out_hbm,
             idx0, idx1, rows0, rows1, out0, out1,
             sem0, sem1, osem0, osem1):
    wid = lax.axis_index("s") * NC + lax.axis_index("c")

    bufs = ((idx0, rows0, out0, sem0, osem0),
            (idx1, rows1, out1, sem1, osem1))

    def stage(i, buf):
        idx_v, rows_v, _, sem, _ = bufs[buf]
        pltpu.sync_copy(seq_hbm.at[wid * ITERS + i], idx_v)
        for j in range(NSUB):
            pltpu.async_copy(
                w_hbm.at[idx_v.at[j]],
                rows_v.at[pl.ds(j * IDX_CHUNK, IDX_CHUNK)],
                sem,
            )

    def drain(buf):
        idx_v, rows_v, _, sem, _ = bufs[buf]
        for j in range(NSUB):
            pltpu.make_async_copy(
                w_hbm.at[idx_v.at[j]],
                rows_v.at[pl.ds(j * IDX_CHUNK, IDX_CHUNK)],
                sem,
            ).wait()

    def compute(i, buf, first):
        idx_v, rows_v, out_v, _, osem = bufs[buf]
        blk = wid * ITERS + i
        @pl.when(jnp.logical_not(first))
        def _():
            pltpu.make_async_copy(
                out_v, out_hbm.at[pl.ds((blk - 2) * G, G)], osem
            ).wait()

        NE = EMB // LANES

        @plsc.parallel_loop(0, G, 1, unroll=2)
        def group(g):
            base = g * T
            s = [jnp.zeros((LANES,), jnp.float32) for _ in range(NE)]
            c = [jnp.zeros((LANES,), jnp.int32) for _ in range(NE)]
            for t in range(T):
                for e in range(NE):
                    r = rows_v[base + t, pl.ds(e * LANES, LANES)]
                    s[e] = s[e] + r
                    b = lax.bitcast_convert_type(r, jnp.int32)
                    c[e] = jnp.where(b != 0, c[e] + 1, c[e])
            for e in range(NE):
                cf = c[e].astype(jnp.float32)
                out_v[g, pl.ds(e * LANES, LANES)] = jnp.where(
                    c[e] == 0, 0.0, s[e] / cf)

        pltpu.async_copy(out_v, out_hbm.at[pl.ds(blk * G, G)], osem)

    stage(0, 0)
    stage(1, 1)

    def outer(io, carry):
        i0 = 2 * io
        drain(0)
        compute(i0, 0, first=io == 0)

        @pl.when(io < NPAIR - 1)
        def _():
            stage(i0 + 2, 0)

        drain(1)
        compute(i0 + 1, 1, first=io == 0)

        @pl.when(io < NPAIR - 1)
        def _():
            stage(i0 + 3, 1)

        return carry

    lax.fori_loop(0, NPAIR, outer, 0)
    for buf in range(2):
        _, _, out_v, _, osem = bufs[buf]
        pltpu.make_async_copy(
            out_v,
            out_hbm.at[pl.ds((wid * ITERS + ITERS - 2 + buf) * G, G)],
            osem,
        ).wait()


def kernel(seq, W):
    w_full = jnp.concatenate([jnp.zeros((1, EMB), W.dtype), W], axis=0)
    seq3 = seq.reshape(ROWS // G, NSUB, IDX_CHUNK)
    mesh = plsc.VectorSubcoreMesh(core_axis_name="c", subcore_axis_name="s")
    out = pl.kernel(
        _sc_body,
        mesh=mesh,
        compiler_params=pltpu.CompilerParams(use_tc_tiling_on_sc=False),
        out_type=jax.ShapeDtypeStruct((ROWS, EMB), jnp.float32),
        scratch_types=[
            pltpu.VMEM((NSUB, IDX_CHUNK), jnp.int32),
            pltpu.VMEM((NSUB, IDX_CHUNK), jnp.int32),
            pltpu.VMEM((IDX_PER_ITER, EMB), jnp.float32),
            pltpu.VMEM((IDX_PER_ITER, EMB), jnp.float32),
            pltpu.VMEM((G, EMB), jnp.float32),
            pltpu.VMEM((G, EMB), jnp.float32),
            pltpu.SemaphoreType.DMA,
            pltpu.SemaphoreType.DMA,
            pltpu.SemaphoreType.DMA,
            pltpu.SemaphoreType.DMA,
        ],
    )(seq3, w_full)
    return out.reshape(B, LSEQ, EMB)
